# Initial kernel scaffold; baseline (speedup 1.0000x reference)
import jax
import jax.numpy as jnp
from jax.experimental import pallas as pl


def kernel(x, edge_index, Ws, a_srcs, a_dsts, biases, W_out, a_src_out, a_dst_out, bias_out):
    raise NotImplementedError("write your pallas kernel here")


# speedup vs baseline: 27.7681x; 27.7681x over previous
import functools

import jax
import jax.numpy as jnp
from jax import lax
from jax.experimental import pallas as pl
from jax.experimental.pallas import tpu as pltpu
from jax.experimental.pallas import tpu_sc as plsc

N = 10000
NP = 10240
E = 320000
D = 128
HID = 16
OUT = 64
HEADS = 8

B = 1024
NB = NP // B
NSUB = 16
EP = E // NSUB
KE = 80
NCH = EP // KE
RPT = NP // NSUB
RCH = 64

f32 = jnp.float32
i32 = jnp.int32

_SC_PARAMS = pltpu.CompilerParams(use_tc_tiling_on_sc=False,
                                  needs_layout_passes=False)


def _leaky(t):
    return jnp.where(t >= 0, t, t * 0.2)


def _tk1_body(nchunk, x_ref, w_ref, ams_ref, amd_ref, *rest):
    hrefs = rest[:nchunk]
    asT, adT, gmax = rest[nchunk:]
    h = jnp.dot(x_ref[...], w_ref[...], preferred_element_type=f32)
    cw = h.shape[1] // nchunk
    for q, href in enumerate(hrefs):
        href[...] = h[:, q * cw:(q + 1) * cw]
    a_s = jnp.dot(h, ams_ref[...], preferred_element_type=f32)
    asT[...] = lax.dot_general(ams_ref[...], h, (((0,), (1,)), ((), ())),
                               preferred_element_type=f32)
    adT[...] = lax.dot_general(amd_ref[...], h, (((0,), (1,)), ((), ())),
                               preferred_element_type=f32)
    bm = jnp.broadcast_to(jnp.max(a_s, axis=0)[:, None], (HEADS, 128))
    @pl.when(pl.program_id(0) == 0)
    def _():
        gmax[...] = bm
    @pl.when(pl.program_id(0) != 0)
    def _():
        gmax[...] = jnp.maximum(gmax[...], bm)


def _tk1(x, w, ams, amd, din, dcat, nchunk):
    hchunk = dcat // nchunk
    body = functools.partial(_tk1_body, nchunk)
    return pl.pallas_call(
        body,
        grid=(NB,),
        in_specs=[
            pl.BlockSpec((B, din), lambda i: (i, 0)),
            pl.BlockSpec((din, dcat), lambda i: (0, 0)),
            pl.BlockSpec((dcat, HEADS), lambda i: (0, 0)),
            pl.BlockSpec((dcat, HEADS), lambda i: (0, 0)),
        ],
        out_specs=[pl.BlockSpec((B, hchunk), lambda i: (i, 0))] * nchunk + [
            pl.BlockSpec((HEADS, B), lambda i: (0, i)),
            pl.BlockSpec((HEADS, B), lambda i: (0, i)),
            pl.BlockSpec((HEADS, 128), lambda i: (0, 0)),
        ],
        out_shape=[jax.ShapeDtypeStruct((NP, hchunk), f32)] * nchunk + [
            jax.ShapeDtypeStruct((HEADS, NP), f32),
            jax.ShapeDtypeStruct((HEADS, NP), f32),
            jax.ShapeDtypeStruct((HEADS, 128), f32),
        ],
    )(x, w, ams, amd)


def _tk2_body(asT_ref, adT_ref, gmax_ref, cT, wsT):
    g = gmax_ref[...][:, 0:1]
    a_s = asT_ref[...]
    a_d = adT_ref[...]
    c = _leaky(g + a_d)
    cT[...] = c
    wsT[...] = jnp.exp(_leaky(a_s + a_d) - c)


def _tk2(asT, adT, gmax):
    return pl.pallas_call(
        _tk2_body,
        out_shape=[jax.ShapeDtypeStruct((HEADS, NP), f32),
                   jax.ShapeDtypeStruct((HEADS, NP), f32)],
    )(asT, adT, gmax)


def _sc_body(C, hpc, esrc, edst, asT, adT, cT, *rest):
    nq = HEADS // hpc
    ppc = nq // 2
    hrefs = rest[:nq]
    orefs = rest[nq:2 * nq]
    drefs = rest[2 * nq:3 * nq]
    (tabs, tabd, tabc, rows, wpad, srcb, dstb, vbuf, vbuf16,
     out_sp, den_sp, sem) = rest[3 * nq:]
    core = lax.axis_index("c")
    sid = lax.axis_index("s")
    G = C // 16
    row0 = sid * RPT

    @pl.loop(0, KE)
    def _(e):
        wpad[e, pl.ds(0, 16)] = jnp.zeros((16,), f32)

    for p in range(ppc):
        for cc in range(2):
            q = ppc * cc + p

            @pl.when(core == cc)
            def _(q=q):
                h_hbm = hrefs[q]
                o_hbm = orefs[q]
                d_hbm = drefs[q]
                j0 = hpc * q
                pltpu.sync_copy(asT.at[pl.ds(j0, hpc)], tabs)
                pltpu.sync_copy(adT.at[pl.ds(j0, hpc)], tabd)
                pltpu.sync_copy(cT.at[pl.ds(j0, hpc)], tabc)

                @pl.loop(0, RCH)
                def _(r):
                    for g in range(G):
                        vbuf[r, pl.ds(g * 16, 16)] = jnp.zeros((16,), f32)
                    vbuf16[r, pl.ds(0, 16)] = jnp.zeros((16,), f32)
                for t in range(RPT // RCH):
                    pltpu.sync_copy(vbuf, out_sp.at[pl.ds(row0 + t * RCH, RCH)])
                    pltpu.sync_copy(vbuf16, den_sp.at[pl.ds(row0 + t * RCH, RCH)])
                plsc.subcore_barrier()

                @pl.loop(0, NCH)
                def _(k):
                    base = sid * EP + k * KE
                    pltpu.sync_copy(esrc.at[pl.ds(base, KE)], srcb)
                    pltpu.sync_copy(edst.at[pl.ds(base, KE)], dstb)
                    cp = pltpu.async_copy(h_hbm.at[srcb], rows, sem)
                    for v in range(KE // 16):
                        s16 = srcb[pl.ds(v * 16, 16)]
                        d16 = dstb[pl.ds(v * 16, 16)]
                        rowi = jnp.arange(16, dtype=i32) + (v * 16)
                        for dlt in range(hpc):
                            dv = jnp.full((16,), dlt, i32)
                            a_s = plsc.load_gather(tabs, [dv, s16])
                            a_d = plsc.load_gather(tabd, [dv, d16])
                            cb = plsc.load_gather(tabc, [dv, d16])
                            w = jnp.exp(_leaky(a_s + a_d) - cb)
                            plsc.store_scatter(wpad, [rowi, dv], w)
                    cp.wait()
                    gph = G // hpc
                    @pl.loop(0, KE)
                    def _(e):
                        wv = wpad[e, pl.ds(0, 16)]
                        for g in range(G):
                            wj = wv[g // gph]
                            sl = pl.ds(g * 16, 16)
                            rows[e, sl] = rows[e, sl] * wj
                    pltpu.sync_copy(rows, out_sp.at[dstb], add=True)
                    pltpu.sync_copy(wpad, den_sp.at[dstb], add=True)

                plsc.subcore_barrier()
                for t in range(RPT // RCH):
                    sl = pl.ds(row0 + t * RCH, RCH)
                    pltpu.sync_copy(out_sp.at[sl], vbuf)
                    pltpu.sync_copy(vbuf, o_hbm.at[sl])
                    pltpu.sync_copy(den_sp.at[sl], vbuf16)
                    pltpu.sync_copy(vbuf16, d_hbm.at[sl])


def _sc_layer(C, hpc, esrc, edst, asT, adT, cT, hs):
    nq = HEADS // hpc
    mesh = plsc.VectorSubcoreMesh(core_axis_name="c", subcore_axis_name="s")
    kern = pl.kernel(
        functools.partial(_sc_body, C, hpc),
        out_type=[jax.ShapeDtypeStruct((NP, C), f32)] * nq +
                 [jax.ShapeDtypeStruct((NP, 16), f32)] * nq,
        mesh=mesh,
        compiler_params=_SC_PARAMS,
        scratch_types=[
            pltpu.VMEM((hpc, NP), f32),
            pltpu.VMEM((hpc, NP), f32),
            pltpu.VMEM((hpc, NP), f32),
            pltpu.VMEM((KE, C), f32),
            pltpu.VMEM((KE, 16), f32),
            pltpu.VMEM((KE,), i32),
            pltpu.VMEM((KE,), i32),
            pltpu.VMEM((RCH, C), f32),
            pltpu.VMEM((RCH, 16), f32),
            pltpu.VMEM_SHARED((NP, C), f32),
            pltpu.VMEM_SHARED((NP, 16), f32),
            pltpu.SemaphoreType.DMA,
        ],
    )
    res = kern(esrc, edst, asT, adT, cT, *hs)
    return res[:nq], res[nq:]


def _tk3_body(*refs):
    (a0, a1, a2, a3, e0, e1, e2, e3, ws_ref,
     g0, g1, g2, g3, w2_ref, ams_ref, amd_ref, b1_ref) = refs[:17]
    orefs = refs[17:25]
    asT, adT, gmax = refs[25:]
    ws = ws_ref[...]
    accs = (a0, a1, a2, a3)
    dens = (e0, e1, e2, e3)
    hs = (g0, g1, g2, g3)
    pieces = []
    for q in range(4):
        acc = accs[q][...]
        den = dens[q][...]
        hq = hs[q][...]
        for dlt in range(2):
            j = 2 * q + dlt
            wsj = ws[j][:, None]
            num = acc[:, dlt * HID:(dlt + 1) * HID] + wsj * hq[:, dlt * HID:(dlt + 1) * HID]
            dj = den[:, dlt][:, None] + wsj + 1e-16
            pieces.append(num / dj)
    x1 = jnp.concatenate(pieces, axis=1) + b1_ref[...]
    h2 = jnp.dot(x1, w2_ref[...], preferred_element_type=f32)
    for q, oref in enumerate(orefs):
        oref[...] = h2[:, q * OUT:(q + 1) * OUT]
    a_s = jnp.dot(h2, ams_ref[...], preferred_element_type=f32)
    asT[...] = lax.dot_general(ams_ref[...], h2, (((0,), (1,)), ((), ())),
                               preferred_element_type=f32)
    adT[...] = lax.dot_general(amd_ref[...], h2, (((0,), (1,)), ((), ())),
                               preferred_element_type=f32)
    bm = jnp.broadcast_to(jnp.max(a_s, axis=0)[:, None], (HEADS, 128))
    @pl.when(pl.program_id(0) == 0)
    def _():
        gmax[...] = bm
    @pl.when(pl.program_id(0) != 0)
    def _():
        gmax[...] = jnp.maximum(gmax[...], bm)


def _tk3(acc1, den1, wsT1, h4, w2, ams2, amd2, b1):
    nblk = lambda s: pl.BlockSpec(s, lambda i: (i, 0))
    tblk = pl.BlockSpec((HEADS, B), lambda i: (0, i))
    return pl.pallas_call(
        _tk3_body,
        grid=(NB,),
        in_specs=[nblk((B, 32))] * 4 + [nblk((B, 16))] * 4 + [tblk] +
                 [nblk((B, 32))] * 4 + [
            pl.BlockSpec((D, 512), lambda i: (0, 0)),
            pl.BlockSpec((512, HEADS), lambda i: (0, 0)),
            pl.BlockSpec((512, HEADS), lambda i: (0, 0)),
            pl.BlockSpec((1, D), lambda i: (0, 0)),
        ],
        out_specs=[nblk((B, OUT))] * 8 + [tblk, tblk,
                   pl.BlockSpec((HEADS, 128), lambda i: (0, 0))],
        out_shape=[jax.ShapeDtypeStruct((NP, OUT), f32)] * 8 + [
            jax.ShapeDtypeStruct((HEADS, NP), f32),
            jax.ShapeDtypeStruct((HEADS, NP), f32),
            jax.ShapeDtypeStruct((HEADS, 128), f32),
        ],
    )(*acc1, *den1, wsT1, *h4, w2, ams2, amd2, b1)


def _tk5_body(*refs):
    accs = refs[:8]
    dens = refs[8:16]
    ws_ref = refs[16]
    hs = refs[17:25]
    bo_ref = refs[25]
    out = refs[26]
    ws = ws_ref[...]
    tot = jnp.zeros((B, OUT), f32)
    for j in range(HEADS):
        acc = accs[j][...]
        den = dens[j][...]
        hj = hs[j][...]
        wsj = ws[j][:, None]
        num = acc + wsj * hj
        dj = den[:, 0][:, None] + wsj + 1e-16
        tot = tot + num / dj
    y = tot * (1.0 / HEADS) + bo_ref[...]
    y = jnp.where(y > 0, y, jnp.exp(jnp.minimum(y, 0.0)) - 1.0)
    m = jnp.max(y, axis=1, keepdims=True)
    z = y - m
    out[...] = z - jnp.log(jnp.sum(jnp.exp(z), axis=1, keepdims=True))


def _tk5(acc2, den2, wsT2, h2c, bo):
    nblk = lambda s: pl.BlockSpec(s, lambda i: (i, 0))
    tblk = pl.BlockSpec((HEADS, B), lambda i: (0, i))
    return pl.pallas_call(
        _tk5_body,
        grid=(NB,),
        in_specs=[nblk((B, OUT))] * 8 + [nblk((B, 16))] * 8 + [tblk] +
                 [nblk((B, OUT))] * 8 + [pl.BlockSpec((1, OUT), lambda i: (0, 0))],
        out_specs=nblk((B, OUT)),
        out_shape=jax.ShapeDtypeStruct((NP, OUT), f32),
    )(*acc2, *den2, wsT2, *h2c, bo)


def kernel(x, edge_index, Ws, a_srcs, a_dsts, biases, W_out,
           a_src_out, a_dst_out, bias_out):
    x = x.astype(f32)
    xp = jnp.pad(x, ((0, NP - N), (0, 0)))
    esrc = edge_index[0].astype(i32)
    edst = edge_index[1].astype(i32)

    eye8 = jnp.eye(HEADS, dtype=f32)
    W1 = Ws.astype(f32).transpose(1, 0, 2).reshape(D, HEADS * HID)
    ams1 = (eye8[:, None, :] * a_srcs.astype(f32)[:, :, None]).reshape(HEADS * HID, HEADS)
    amd1 = (eye8[:, None, :] * a_dsts.astype(f32)[:, :, None]).reshape(HEADS * HID, HEADS)
    ams2 = (eye8[:, None, :] * a_src_out.astype(f32)[:, :, None]).reshape(HEADS * OUT, HEADS)
    amd2 = (eye8[:, None, :] * a_dst_out.astype(f32)[:, :, None]).reshape(HEADS * OUT, HEADS)
    b1 = biases.astype(f32).reshape(1, HEADS * HID)
    bo = bias_out.astype(f32).reshape(1, OUT)

    *h4_1, asT1, adT1, gmax1 = _tk1(xp, W1, ams1, amd1, D, HEADS * HID, 4)
    cT1, wsT1 = _tk2(asT1, adT1, gmax1)
    acc1, den1 = _sc_layer(32, 2, esrc, edst, asT1, adT1, cT1, h4_1)

    *h2c, asT2, adT2, gmax2 = _tk3(acc1, den1, wsT1, h4_1, W_out.astype(f32),
                                   ams2, amd2, b1)
    cT2, wsT2 = _tk2(asT2, adT2, gmax2)
    acc2, den2 = _sc_layer(OUT, 1, esrc, edst, asT2, adT2, cT2, h2c)

    out = _tk5(acc2, den2, wsT2, h2c, bo)
    return out[:N]


# speedup vs baseline: 45.6421x; 1.6437x over previous
import functools

import jax
import jax.numpy as jnp
from jax import lax
from jax.experimental import pallas as pl
from jax.experimental.pallas import tpu as pltpu
from jax.experimental.pallas import tpu_sc as plsc

N = 10000
NP = 10240
E = 320000
D = 128
HID = 16
OUT = 64
HEADS = 8

B = 1024
NB = NP // B
NSUB = 16
EP = E // NSUB
KE = 80
NCH = EP // KE
RPT = NP // NSUB
RCH = 64

f32 = jnp.float32
i32 = jnp.int32

_SC_PARAMS = pltpu.CompilerParams(use_tc_tiling_on_sc=False,
                                  needs_layout_passes=False)


def _leaky(t):
    return jnp.where(t >= 0, t, t * 0.2)


def _tk1_body(nchunk, x_ref, w_ref, ams_ref, amd_ref, *rest):
    hrefs = rest[:nchunk]
    asT, adT, gmax = rest[nchunk:]
    h = jnp.dot(x_ref[...], w_ref[...], preferred_element_type=f32)
    cw = h.shape[1] // nchunk
    for q, href in enumerate(hrefs):
        href[...] = h[:, q * cw:(q + 1) * cw]
    a_s = jnp.dot(h, ams_ref[...], preferred_element_type=f32)
    asT[...] = lax.dot_general(ams_ref[...], h, (((0,), (1,)), ((), ())),
                               preferred_element_type=f32)
    adT[...] = lax.dot_general(amd_ref[...], h, (((0,), (1,)), ((), ())),
                               preferred_element_type=f32)
    bm = jnp.broadcast_to(jnp.max(a_s, axis=0)[:, None], (HEADS, 128))
    @pl.when(pl.program_id(0) == 0)
    def _():
        gmax[...] = bm
    @pl.when(pl.program_id(0) != 0)
    def _():
        gmax[...] = jnp.maximum(gmax[...], bm)


def _tk1(x, w, ams, amd, din, dcat, nchunk):
    hchunk = dcat // nchunk
    body = functools.partial(_tk1_body, nchunk)
    return pl.pallas_call(
        body,
        grid=(NB,),
        in_specs=[
            pl.BlockSpec((B, din), lambda i: (i, 0)),
            pl.BlockSpec((din, dcat), lambda i: (0, 0)),
            pl.BlockSpec((dcat, HEADS), lambda i: (0, 0)),
            pl.BlockSpec((dcat, HEADS), lambda i: (0, 0)),
        ],
        out_specs=[pl.BlockSpec((B, hchunk), lambda i: (i, 0))] * nchunk + [
            pl.BlockSpec((HEADS, B), lambda i: (0, i)),
            pl.BlockSpec((HEADS, B), lambda i: (0, i)),
            pl.BlockSpec((HEADS, 128), lambda i: (0, 0)),
        ],
        out_shape=[jax.ShapeDtypeStruct((NP, hchunk), f32)] * nchunk + [
            jax.ShapeDtypeStruct((HEADS, NP), f32),
            jax.ShapeDtypeStruct((HEADS, NP), f32),
            jax.ShapeDtypeStruct((HEADS, 128), f32),
        ],
    )(x, w, ams, amd)


def _tk2_body(asT_ref, adT_ref, gmax_ref, cT, wsT):
    g = gmax_ref[...][:, 0:1]
    a_s = asT_ref[...]
    a_d = adT_ref[...]
    c = _leaky(g + a_d)
    cT[...] = c
    wsT[...] = jnp.exp(_leaky(a_s + a_d) - c)


def _tk2(asT, adT, gmax):
    return pl.pallas_call(
        _tk2_body,
        out_shape=[jax.ShapeDtypeStruct((HEADS, NP), f32),
                   jax.ShapeDtypeStruct((HEADS, NP), f32)],
    )(asT, adT, gmax)


def _sc_body(C, hpc, epk, asT, adT, cT, *rest):
    nq = HEADS // hpc
    ppc = nq // 2
    hrefs = rest[:nq]
    orefs = rest[nq:2 * nq]
    drefs = rest[2 * nq:3 * nq]
    (tabs, tabd, tabc, rows2, wpad2, sd2, vbuf, vbuf16,
     out_sp, den_sp, gsem0, gsem1, ssem0, ssem1) = rest[3 * nq:]
    gsem = (gsem0, gsem1)
    ssem = (ssem0, ssem1)
    core = lax.axis_index("c")
    sid = lax.axis_index("s")
    G = C // 16
    gph = G // hpc
    row0 = sid * RPT

    @pl.loop(0, KE)
    def _(e):
        for b in range(2):
            wpad2[b, e, pl.ds(0, 16)] = jnp.zeros((16,), f32)

    for p in range(ppc):
        for cc in range(2):
            q = ppc * cc + p

            @pl.when(core == cc)
            def _(q=q):
                h_hbm = hrefs[q]
                o_hbm = orefs[q]
                d_hbm = drefs[q]
                j0 = hpc * q
                pltpu.sync_copy(asT.at[pl.ds(j0, hpc)], tabs)
                pltpu.sync_copy(adT.at[pl.ds(j0, hpc)], tabd)
                pltpu.sync_copy(cT.at[pl.ds(j0, hpc)], tabc)

                @pl.loop(0, RCH)
                def _(r):
                    for g in range(G):
                        vbuf[r, pl.ds(g * 16, 16)] = jnp.zeros((16,), f32)
                    vbuf16[r, pl.ds(0, 16)] = jnp.zeros((16,), f32)
                for t in range(RPT // RCH):
                    pltpu.sync_copy(vbuf, out_sp.at[pl.ds(row0 + t * RCH, RCH)])
                    pltpu.sync_copy(vbuf16, den_sp.at[pl.ds(row0 + t * RCH, RCH)])
                plsc.subcore_barrier()

                def drain_gather(b):
                    pltpu.make_async_copy(h_hbm.at[pl.ds(0, KE)],
                                          rows2.at[b], gsem[b]).wait()

                def drain_scatter(b):
                    pltpu.make_async_copy(h_hbm.at[pl.ds(0, KE)],
                                          rows2.at[b], ssem[b]).wait()
                    pltpu.make_async_copy(d_hbm.at[pl.ds(0, KE)],
                                          wpad2.at[b], ssem[b]).wait()

                cid0 = sid * NCH
                pltpu.sync_copy(epk.at[cid0], sd2.at[0])
                pltpu.async_copy(h_hbm.at[sd2.at[0, 0]], rows2.at[0], gsem[0])

                @pl.loop(0, NCH // 2)
                def _(k2):
                    for b in range(2):
                        nb = 1 - b
                        k = k2 * 2 + b
                        @pl.when(k + 1 < NCH)
                        def _():
                            @pl.when(k >= 1)
                            def _():
                                drain_scatter(nb)
                            pltpu.sync_copy(epk.at[cid0 + k + 1], sd2.at[nb])
                            pltpu.async_copy(h_hbm.at[sd2.at[nb, 0]],
                                             rows2.at[nb], gsem[nb])
                        for v in range(KE // 16):
                            s16 = sd2[b, 0, pl.ds(v * 16, 16)]
                            d16 = sd2[b, 1, pl.ds(v * 16, 16)]
                            rowi = jnp.arange(16, dtype=i32) + (v * 16)
                            for dlt in range(hpc):
                                dv = jnp.full((16,), dlt, i32)
                                a_s = plsc.load_gather(tabs, [dv, s16])
                                a_d = plsc.load_gather(tabd, [dv, d16])
                                cb = plsc.load_gather(tabc, [dv, d16])
                                w = jnp.exp(_leaky(a_s + a_d) - cb)
                                plsc.store_scatter(wpad2.at[b], [rowi, dv], w)
                        drain_gather(b)
                        @pl.loop(0, KE)
                        def _(e):
                            wv = wpad2[b, e, pl.ds(0, 16)]
                            for g in range(G):
                                wj = wv[g // gph]
                                sl = pl.ds(g * 16, 16)
                                rows2[b, e, sl] = rows2[b, e, sl] * wj
                        pltpu.async_copy(rows2.at[b], out_sp.at[sd2.at[b, 1]],
                                         ssem[b], add=True)
                        pltpu.async_copy(wpad2.at[b], den_sp.at[sd2.at[b, 1]],
                                         ssem[b], add=True)

                drain_scatter(0)
                drain_scatter(1)

                plsc.subcore_barrier()
                for t in range(RPT // RCH):
                    sl = pl.ds(row0 + t * RCH, RCH)
                    pltpu.sync_copy(out_sp.at[sl], vbuf)
                    pltpu.sync_copy(vbuf, o_hbm.at[sl])
                    pltpu.sync_copy(den_sp.at[sl], vbuf16)
                    pltpu.sync_copy(vbuf16, d_hbm.at[sl])


def _sc_layer(C, hpc, epk, asT, adT, cT, hs):
    nq = HEADS // hpc
    mesh = plsc.VectorSubcoreMesh(core_axis_name="c", subcore_axis_name="s")
    kern = pl.kernel(
        functools.partial(_sc_body, C, hpc),
        out_type=[jax.ShapeDtypeStruct((NP, C), f32)] * nq +
                 [jax.ShapeDtypeStruct((NP, 16), f32)] * nq,
        mesh=mesh,
        compiler_params=_SC_PARAMS,
        scratch_types=[
            pltpu.VMEM((hpc, NP), f32),
            pltpu.VMEM((hpc, NP), f32),
            pltpu.VMEM((hpc, NP), f32),
            pltpu.VMEM((2, KE, C), f32),
            pltpu.VMEM((2, KE, 16), f32),
            pltpu.VMEM((2, 2, KE), i32),
            pltpu.VMEM((RCH, C), f32),
            pltpu.VMEM((RCH, 16), f32),
            pltpu.VMEM_SHARED((NP, C), f32),
            pltpu.VMEM_SHARED((NP, 16), f32),
            pltpu.SemaphoreType.DMA,
            pltpu.SemaphoreType.DMA,
            pltpu.SemaphoreType.DMA,
            pltpu.SemaphoreType.DMA,
        ],
    )
    res = kern(epk, asT, adT, cT, *hs)
    return res[:nq], res[nq:]


def _tk3_body(*refs):
    (a0, a1, a2, a3, e0, e1, e2, e3, ws_ref,
     g0, g1, g2, g3, w2_ref, ams_ref, amd_ref, b1_ref) = refs[:17]
    orefs = refs[17:25]
    asT, adT, gmax = refs[25:]
    ws = ws_ref[...]
    accs = (a0, a1, a2, a3)
    dens = (e0, e1, e2, e3)
    hs = (g0, g1, g2, g3)
    pieces = []
    for q in range(4):
        acc = accs[q][...]
        den = dens[q][...]
        hq = hs[q][...]
        for dlt in range(2):
            j = 2 * q + dlt
            wsj = ws[j][:, None]
            num = acc[:, dlt * HID:(dlt + 1) * HID] + wsj * hq[:, dlt * HID:(dlt + 1) * HID]
            dj = den[:, dlt][:, None] + wsj + 1e-16
            pieces.append(num / dj)
    x1 = jnp.concatenate(pieces, axis=1) + b1_ref[...]
    h2 = jnp.dot(x1, w2_ref[...], preferred_element_type=f32)
    for q, oref in enumerate(orefs):
        oref[...] = h2[:, q * OUT:(q + 1) * OUT]
    a_s = jnp.dot(h2, ams_ref[...], preferred_element_type=f32)
    asT[...] = lax.dot_general(ams_ref[...], h2, (((0,), (1,)), ((), ())),
                               preferred_element_type=f32)
    adT[...] = lax.dot_general(amd_ref[...], h2, (((0,), (1,)), ((), ())),
                               preferred_element_type=f32)
    bm = jnp.broadcast_to(jnp.max(a_s, axis=0)[:, None], (HEADS, 128))
    @pl.when(pl.program_id(0) == 0)
    def _():
        gmax[...] = bm
    @pl.when(pl.program_id(0) != 0)
    def _():
        gmax[...] = jnp.maximum(gmax[...], bm)


def _tk3(acc1, den1, wsT1, h4, w2, ams2, amd2, b1):
    nblk = lambda s: pl.BlockSpec(s, lambda i: (i, 0))
    tblk = pl.BlockSpec((HEADS, B), lambda i: (0, i))
    return pl.pallas_call(
        _tk3_body,
        grid=(NB,),
        in_specs=[nblk((B, 32))] * 4 + [nblk((B, 16))] * 4 + [tblk] +
                 [nblk((B, 32))] * 4 + [
            pl.BlockSpec((D, 512), lambda i: (0, 0)),
            pl.BlockSpec((512, HEADS), lambda i: (0, 0)),
            pl.BlockSpec((512, HEADS), lambda i: (0, 0)),
            pl.BlockSpec((1, D), lambda i: (0, 0)),
        ],
        out_specs=[nblk((B, OUT))] * 8 + [tblk, tblk,
                   pl.BlockSpec((HEADS, 128), lambda i: (0, 0))],
        out_shape=[jax.ShapeDtypeStruct((NP, OUT), f32)] * 8 + [
            jax.ShapeDtypeStruct((HEADS, NP), f32),
            jax.ShapeDtypeStruct((HEADS, NP), f32),
            jax.ShapeDtypeStruct((HEADS, 128), f32),
        ],
    )(*acc1, *den1, wsT1, *h4, w2, ams2, amd2, b1)


def _tk5_body(*refs):
    accs = refs[:8]
    dens = refs[8:16]
    ws_ref = refs[16]
    hs = refs[17:25]
    bo_ref = refs[25]
    out = refs[26]
    ws = ws_ref[...]
    tot = jnp.zeros((B, OUT), f32)
    for j in range(HEADS):
        acc = accs[j][...]
        den = dens[j][...]
        hj = hs[j][...]
        wsj = ws[j][:, None]
        num = acc + wsj * hj
        dj = den[:, 0][:, None] + wsj + 1e-16
        tot = tot + num / dj
    y = tot * (1.0 / HEADS) + bo_ref[...]
    y = jnp.where(y > 0, y, jnp.exp(jnp.minimum(y, 0.0)) - 1.0)
    m = jnp.max(y, axis=1, keepdims=True)
    z = y - m
    out[...] = z - jnp.log(jnp.sum(jnp.exp(z), axis=1, keepdims=True))


def _tk5(acc2, den2, wsT2, h2c, bo):
    nblk = lambda s: pl.BlockSpec(s, lambda i: (i, 0))
    tblk = pl.BlockSpec((HEADS, B), lambda i: (0, i))
    return pl.pallas_call(
        _tk5_body,
        grid=(NB,),
        in_specs=[nblk((B, OUT))] * 8 + [nblk((B, 16))] * 8 + [tblk] +
                 [nblk((B, OUT))] * 8 + [pl.BlockSpec((1, OUT), lambda i: (0, 0))],
        out_specs=nblk((B, OUT)),
        out_shape=jax.ShapeDtypeStruct((NP, OUT), f32),
    )(*acc2, *den2, wsT2, *h2c, bo)


def kernel(x, edge_index, Ws, a_srcs, a_dsts, biases, W_out,
           a_src_out, a_dst_out, bias_out):
    x = x.astype(f32)
    xp = jnp.pad(x, ((0, NP - N), (0, 0)))
    epk = edge_index.astype(i32).reshape(2, E // KE, KE).transpose(1, 0, 2)

    eye8 = jnp.eye(HEADS, dtype=f32)
    W1 = Ws.astype(f32).transpose(1, 0, 2).reshape(D, HEADS * HID)
    ams1 = (eye8[:, None, :] * a_srcs.astype(f32)[:, :, None]).reshape(HEADS * HID, HEADS)
    amd1 = (eye8[:, None, :] * a_dsts.astype(f32)[:, :, None]).reshape(HEADS * HID, HEADS)
    ams2 = (eye8[:, None, :] * a_src_out.astype(f32)[:, :, None]).reshape(HEADS * OUT, HEADS)
    amd2 = (eye8[:, None, :] * a_dst_out.astype(f32)[:, :, None]).reshape(HEADS * OUT, HEADS)
    b1 = biases.astype(f32).reshape(1, HEADS * HID)
    bo = bias_out.astype(f32).reshape(1, OUT)

    *h4_1, asT1, adT1, gmax1 = _tk1(xp, W1, ams1, amd1, D, HEADS * HID, 4)
    cT1, wsT1 = _tk2(asT1, adT1, gmax1)
    acc1, den1 = _sc_layer(32, 2, epk, asT1, adT1, cT1, h4_1)

    *h2c, asT2, adT2, gmax2 = _tk3(acc1, den1, wsT1, h4_1, W_out.astype(f32),
                                   ams2, amd2, b1)
    cT2, wsT2 = _tk2(asT2, adT2, gmax2)
    acc2, den2 = _sc_layer(OUT, 1, epk, asT2, adT2, cT2, h2c)

    out = _tk5(acc2, den2, wsT2, h2c, bo)
    return out[:N]


# speedup vs baseline: 49.2268x; 1.0785x over previous
import functools

import jax
import jax.numpy as jnp
from jax import lax
from jax.experimental import pallas as pl
from jax.experimental.pallas import tpu as pltpu
from jax.experimental.pallas import tpu_sc as plsc

N = 10000
NP = 10240
E = 320000
D = 128
HID = 16
OUT = 64
HEADS = 8

B = 1024
NB = NP // B
NSUB = 16
EP = E // NSUB
KE = 80
NCH = EP // KE
RPT = NP // NSUB
RCH = 64

f32 = jnp.float32
i32 = jnp.int32

_SC_PARAMS = pltpu.CompilerParams(use_tc_tiling_on_sc=False,
                                  needs_layout_passes=False)


def _leaky(t):
    return jnp.where(t >= 0, t, t * 0.2)


def _tk1_body(nchunk, x_ref, w_ref, ams_ref, amd_ref, *rest):
    hrefs = rest[:nchunk]
    asT, adT, gmax = rest[nchunk:]
    h = jnp.dot(x_ref[...], w_ref[...], preferred_element_type=f32)
    cw = h.shape[1] // nchunk
    for q, href in enumerate(hrefs):
        href[...] = h[:, q * cw:(q + 1) * cw]
    a_s = jnp.dot(h, ams_ref[...], preferred_element_type=f32)
    asT[...] = lax.dot_general(ams_ref[...], h, (((0,), (1,)), ((), ())),
                               preferred_element_type=f32)
    adT[...] = lax.dot_general(amd_ref[...], h, (((0,), (1,)), ((), ())),
                               preferred_element_type=f32)
    bm = jnp.broadcast_to(jnp.max(a_s, axis=0)[:, None], (HEADS, 128))
    @pl.when(pl.program_id(0) == 0)
    def _():
        gmax[...] = bm
    @pl.when(pl.program_id(0) != 0)
    def _():
        gmax[...] = jnp.maximum(gmax[...], bm)


def _tk1(x, w, ams, amd, din, dcat, nchunk):
    hchunk = dcat // nchunk
    body = functools.partial(_tk1_body, nchunk)
    return pl.pallas_call(
        body,
        grid=(NB,),
        in_specs=[
            pl.BlockSpec((B, din), lambda i: (i, 0)),
            pl.BlockSpec((din, dcat), lambda i: (0, 0)),
            pl.BlockSpec((dcat, HEADS), lambda i: (0, 0)),
            pl.BlockSpec((dcat, HEADS), lambda i: (0, 0)),
        ],
        out_specs=[pl.BlockSpec((B, hchunk), lambda i: (i, 0))] * nchunk + [
            pl.BlockSpec((HEADS, B), lambda i: (0, i)),
            pl.BlockSpec((HEADS, B), lambda i: (0, i)),
            pl.BlockSpec((HEADS, 128), lambda i: (0, 0)),
        ],
        out_shape=[jax.ShapeDtypeStruct((NP, hchunk), f32)] * nchunk + [
            jax.ShapeDtypeStruct((HEADS, NP), f32),
            jax.ShapeDtypeStruct((HEADS, NP), f32),
            jax.ShapeDtypeStruct((HEADS, 128), f32),
        ],
    )(x, w, ams, amd)


def _tk2_body(asT_ref, adT_ref, gmax_ref, cT, wsT):
    g = gmax_ref[...][:, 0:1]
    a_s = asT_ref[...]
    a_d = adT_ref[...]
    c = _leaky(g + a_d)
    cT[...] = c
    wsT[...] = jnp.exp(_leaky(a_s + a_d) - c)


def _tk2(asT, adT, gmax):
    return pl.pallas_call(
        _tk2_body,
        out_shape=[jax.ShapeDtypeStruct((HEADS, NP), f32),
                   jax.ShapeDtypeStruct((HEADS, NP), f32)],
    )(asT, adT, gmax)


def _sc_body(C, hpc, epk, asT, adT, cT, *rest):
    nq = HEADS // hpc
    ppc = nq // 2
    hrefs = rest[:nq]
    orefs = rest[nq:2 * nq]
    drefs = rest[2 * nq:3 * nq]
    (tabs, tabd, tabc, rows2, wpad2, sd2, vbuf, vbuf16,
     out_sp, den_sp, gsem0, gsem1, ssem0, ssem1, isem0, isem1) = rest[3 * nq:]
    gsem = (gsem0, gsem1)
    ssem = (ssem0, ssem1)
    isem = (isem0, isem1)
    core = lax.axis_index("c")
    sid = lax.axis_index("s")
    G = C // 16
    gph = G // hpc
    row0 = sid * RPT

    @pl.loop(0, KE)
    def _(e):
        for b in range(2):
            wpad2[b, e, pl.ds(0, 16)] = jnp.zeros((16,), f32)

    for p in range(ppc):
        for cc in range(2):
            q = ppc * cc + p

            @pl.when(core == cc)
            def _(q=q):
                h_hbm = hrefs[q]
                o_hbm = orefs[q]
                d_hbm = drefs[q]
                j0 = hpc * q
                pltpu.sync_copy(asT.at[pl.ds(j0, hpc)], tabs)
                pltpu.sync_copy(adT.at[pl.ds(j0, hpc)], tabd)
                pltpu.sync_copy(cT.at[pl.ds(j0, hpc)], tabc)

                @pl.loop(0, RCH)
                def _(r):
                    for g in range(G):
                        vbuf[r, pl.ds(g * 16, 16)] = jnp.zeros((16,), f32)
                    vbuf16[r, pl.ds(0, 16)] = jnp.zeros((16,), f32)
                for t in range(RPT // RCH):
                    pltpu.sync_copy(vbuf, out_sp.at[pl.ds(row0 + t * RCH, RCH)])
                    pltpu.sync_copy(vbuf16, den_sp.at[pl.ds(row0 + t * RCH, RCH)])
                plsc.subcore_barrier()

                def drain_gather(b):
                    pltpu.make_async_copy(h_hbm.at[pl.ds(0, KE)],
                                          rows2.at[b], gsem[b]).wait()

                def drain_scatter(b):
                    pltpu.make_async_copy(h_hbm.at[pl.ds(0, KE)],
                                          rows2.at[b], ssem[b]).wait()
                    pltpu.make_async_copy(d_hbm.at[pl.ds(0, KE)],
                                          wpad2.at[b], ssem[b]).wait()

                cid0 = sid * NCH
                pltpu.sync_copy(epk.at[cid0], sd2.at[0])
                pltpu.async_copy(h_hbm.at[sd2.at[0, 0]], rows2.at[0], gsem[0])

                @pl.loop(0, NCH // 2)
                def _(k2):
                    for b in range(2):
                        nb = 1 - b
                        k = k2 * 2 + b
                        @pl.when(k + 1 < NCH)
                        def _():
                            @pl.when(k >= 1)
                            def _():
                                drain_scatter(nb)
                            pltpu.async_copy(epk.at[cid0 + k + 1], sd2.at[nb],
                                             isem[nb])
                        for v in range(KE // 16):
                            s16 = sd2[b, 0, pl.ds(v * 16, 16)]
                            d16 = sd2[b, 1, pl.ds(v * 16, 16)]
                            rowi = jnp.arange(16, dtype=i32) + (v * 16)
                            for dlt in range(hpc):
                                dv = jnp.full((16,), dlt, i32)
                                a_s = plsc.load_gather(tabs, [dv, s16])
                                a_d = plsc.load_gather(tabd, [dv, d16])
                                cb = plsc.load_gather(tabc, [dv, d16])
                                w = jnp.exp(_leaky(a_s + a_d) - cb)
                                plsc.store_scatter(wpad2.at[b], [rowi, dv], w)
                        drain_gather(b)
                        @pl.loop(0, KE, unroll=4)
                        def _(e):
                            wv = wpad2[b, e, pl.ds(0, 16)]
                            for g in range(G):
                                wj = wv[g // gph]
                                sl = pl.ds(g * 16, 16)
                                rows2[b, e, sl] = rows2[b, e, sl] * wj
                        pltpu.async_copy(rows2.at[b], out_sp.at[sd2.at[b, 1]],
                                         ssem[b], add=True)
                        pltpu.async_copy(wpad2.at[b], den_sp.at[sd2.at[b, 1]],
                                         ssem[b], add=True)
                        @pl.when(k + 1 < NCH)
                        def _():
                            pltpu.make_async_copy(epk.at[cid0], sd2.at[nb],
                                                  isem[nb]).wait()
                            pltpu.async_copy(h_hbm.at[sd2.at[nb, 0]],
                                             rows2.at[nb], gsem[nb])

                drain_scatter(0)
                drain_scatter(1)

                plsc.subcore_barrier()
                for t in range(RPT // RCH):
                    sl = pl.ds(row0 + t * RCH, RCH)
                    pltpu.sync_copy(out_sp.at[sl], vbuf)
                    pltpu.sync_copy(vbuf, o_hbm.at[sl])
                    pltpu.sync_copy(den_sp.at[sl], vbuf16)
                    pltpu.sync_copy(vbuf16, d_hbm.at[sl])


def _sc_layer(C, hpc, epk, asT, adT, cT, hs):
    nq = HEADS // hpc
    mesh = plsc.VectorSubcoreMesh(core_axis_name="c", subcore_axis_name="s")
    kern = pl.kernel(
        functools.partial(_sc_body, C, hpc),
        out_type=[jax.ShapeDtypeStruct((NP, C), f32)] * nq +
                 [jax.ShapeDtypeStruct((NP, 16), f32)] * nq,
        mesh=mesh,
        compiler_params=_SC_PARAMS,
        scratch_types=[
            pltpu.VMEM((hpc, NP), f32),
            pltpu.VMEM((hpc, NP), f32),
            pltpu.VMEM((hpc, NP), f32),
            pltpu.VMEM((2, KE, C), f32),
            pltpu.VMEM((2, KE, 16), f32),
            pltpu.VMEM((2, 2, KE), i32),
            pltpu.VMEM((RCH, C), f32),
            pltpu.VMEM((RCH, 16), f32),
            pltpu.VMEM_SHARED((NP, C), f32),
            pltpu.VMEM_SHARED((NP, 16), f32),
            pltpu.SemaphoreType.DMA,
            pltpu.SemaphoreType.DMA,
            pltpu.SemaphoreType.DMA,
            pltpu.SemaphoreType.DMA,
            pltpu.SemaphoreType.DMA,
            pltpu.SemaphoreType.DMA,
        ],
    )
    res = kern(epk, asT, adT, cT, *hs)
    return res[:nq], res[nq:]


def _tk3_body(*refs):
    (a0, a1, a2, a3, e0, e1, e2, e3, ws_ref,
     g0, g1, g2, g3, w2_ref, ams_ref, amd_ref, b1_ref) = refs[:17]
    orefs = refs[17:25]
    asT, adT, gmax = refs[25:]
    ws = ws_ref[...]
    accs = (a0, a1, a2, a3)
    dens = (e0, e1, e2, e3)
    hs = (g0, g1, g2, g3)
    pieces = []
    for q in range(4):
        acc = accs[q][...]
        den = dens[q][...]
        hq = hs[q][...]
        for dlt in range(2):
            j = 2 * q + dlt
            wsj = ws[j][:, None]
            num = acc[:, dlt * HID:(dlt + 1) * HID] + wsj * hq[:, dlt * HID:(dlt + 1) * HID]
            dj = den[:, dlt][:, None] + wsj + 1e-16
            pieces.append(num / dj)
    x1 = jnp.concatenate(pieces, axis=1) + b1_ref[...]
    h2 = jnp.dot(x1, w2_ref[...], preferred_element_type=f32)
    for q, oref in enumerate(orefs):
        oref[...] = h2[:, q * OUT:(q + 1) * OUT]
    a_s = jnp.dot(h2, ams_ref[...], preferred_element_type=f32)
    asT[...] = lax.dot_general(ams_ref[...], h2, (((0,), (1,)), ((), ())),
                               preferred_element_type=f32)
    adT[...] = lax.dot_general(amd_ref[...], h2, (((0,), (1,)), ((), ())),
                               preferred_element_type=f32)
    bm = jnp.broadcast_to(jnp.max(a_s, axis=0)[:, None], (HEADS, 128))
    @pl.when(pl.program_id(0) == 0)
    def _():
        gmax[...] = bm
    @pl.when(pl.program_id(0) != 0)
    def _():
        gmax[...] = jnp.maximum(gmax[...], bm)


def _tk3(acc1, den1, wsT1, h4, w2, ams2, amd2, b1):
    nblk = lambda s: pl.BlockSpec(s, lambda i: (i, 0))
    tblk = pl.BlockSpec((HEADS, B), lambda i: (0, i))
    return pl.pallas_call(
        _tk3_body,
        grid=(NB,),
        in_specs=[nblk((B, 32))] * 4 + [nblk((B, 16))] * 4 + [tblk] +
                 [nblk((B, 32))] * 4 + [
            pl.BlockSpec((D, 512), lambda i: (0, 0)),
            pl.BlockSpec((512, HEADS), lambda i: (0, 0)),
            pl.BlockSpec((512, HEADS), lambda i: (0, 0)),
            pl.BlockSpec((1, D), lambda i: (0, 0)),
        ],
        out_specs=[nblk((B, OUT))] * 8 + [tblk, tblk,
                   pl.BlockSpec((HEADS, 128), lambda i: (0, 0))],
        out_shape=[jax.ShapeDtypeStruct((NP, OUT), f32)] * 8 + [
            jax.ShapeDtypeStruct((HEADS, NP), f32),
            jax.ShapeDtypeStruct((HEADS, NP), f32),
            jax.ShapeDtypeStruct((HEADS, 128), f32),
        ],
    )(*acc1, *den1, wsT1, *h4, w2, ams2, amd2, b1)


def _tk5_body(*refs):
    accs = refs[:8]
    dens = refs[8:16]
    ws_ref = refs[16]
    hs = refs[17:25]
    bo_ref = refs[25]
    out = refs[26]
    ws = ws_ref[...]
    tot = jnp.zeros((B, OUT), f32)
    for j in range(HEADS):
        acc = accs[j][...]
        den = dens[j][...]
        hj = hs[j][...]
        wsj = ws[j][:, None]
        num = acc + wsj * hj
        dj = den[:, 0][:, None] + wsj + 1e-16
        tot = tot + num / dj
    y = tot * (1.0 / HEADS) + bo_ref[...]
    y = jnp.where(y > 0, y, jnp.exp(jnp.minimum(y, 0.0)) - 1.0)
    m = jnp.max(y, axis=1, keepdims=True)
    z = y - m
    out[...] = z - jnp.log(jnp.sum(jnp.exp(z), axis=1, keepdims=True))


def _tk5(acc2, den2, wsT2, h2c, bo):
    nblk = lambda s: pl.BlockSpec(s, lambda i: (i, 0))
    tblk = pl.BlockSpec((HEADS, B), lambda i: (0, i))
    return pl.pallas_call(
        _tk5_body,
        grid=(NB,),
        in_specs=[nblk((B, OUT))] * 8 + [nblk((B, 16))] * 8 + [tblk] +
                 [nblk((B, OUT))] * 8 + [pl.BlockSpec((1, OUT), lambda i: (0, 0))],
        out_specs=nblk((B, OUT)),
        out_shape=jax.ShapeDtypeStruct((NP, OUT), f32),
    )(*acc2, *den2, wsT2, *h2c, bo)


def kernel(x, edge_index, Ws, a_srcs, a_dsts, biases, W_out,
           a_src_out, a_dst_out, bias_out):
    x = x.astype(f32)
    xp = jnp.pad(x, ((0, NP - N), (0, 0)))
    epk = edge_index.astype(i32).reshape(2, E // KE, KE).transpose(1, 0, 2)

    eye8 = jnp.eye(HEADS, dtype=f32)
    W1 = Ws.astype(f32).transpose(1, 0, 2).reshape(D, HEADS * HID)
    ams1 = (eye8[:, None, :] * a_srcs.astype(f32)[:, :, None]).reshape(HEADS * HID, HEADS)
    amd1 = (eye8[:, None, :] * a_dsts.astype(f32)[:, :, None]).reshape(HEADS * HID, HEADS)
    ams2 = (eye8[:, None, :] * a_src_out.astype(f32)[:, :, None]).reshape(HEADS * OUT, HEADS)
    amd2 = (eye8[:, None, :] * a_dst_out.astype(f32)[:, :, None]).reshape(HEADS * OUT, HEADS)
    b1 = biases.astype(f32).reshape(1, HEADS * HID)
    bo = bias_out.astype(f32).reshape(1, OUT)

    *h4_1, asT1, adT1, gmax1 = _tk1(xp, W1, ams1, amd1, D, HEADS * HID, 4)
    cT1, wsT1 = _tk2(asT1, adT1, gmax1)
    acc1, den1 = _sc_layer(32, 2, epk, asT1, adT1, cT1, h4_1)

    *h2c, asT2, adT2, gmax2 = _tk3(acc1, den1, wsT1, h4_1, W_out.astype(f32),
                                   ams2, amd2, b1)
    cT2, wsT2 = _tk2(asT2, adT2, gmax2)
    acc2, den2 = _sc_layer(OUT, 1, epk, asT2, adT2, cT2, h2c)

    out = _tk5(acc2, den2, wsT2, h2c, bo)
    return out[:N]


# speedup vs baseline: 49.3777x; 1.0031x over previous
import functools

import jax
import jax.numpy as jnp
from jax import lax
from jax.experimental import pallas as pl
from jax.experimental.pallas import tpu as pltpu
from jax.experimental.pallas import tpu_sc as plsc

N = 10000
NP = 10240
E = 320000
D = 128
HID = 16
OUT = 64
HEADS = 8

B = 1024
NB = NP // B
NSUB = 16
EP = E // NSUB
KE = 80
NCH = EP // KE
RPT = NP // NSUB
RCH = 64

f32 = jnp.float32
i32 = jnp.int32

_SC_PARAMS = pltpu.CompilerParams(use_tc_tiling_on_sc=False,
                                  needs_layout_passes=False)


def _leaky(t):
    return jnp.where(t >= 0, t, t * 0.2)


def _tk1_body(nchunk, x_ref, w_ref, ams_ref, amd_ref, *rest):
    hrefs = rest[:nchunk]
    asT, adT, gmax = rest[nchunk:]
    h = jnp.dot(x_ref[...], w_ref[...], preferred_element_type=f32)
    cw = h.shape[1] // nchunk
    for q, href in enumerate(hrefs):
        href[...] = h[:, q * cw:(q + 1) * cw]
    a_s = jnp.dot(h, ams_ref[...], preferred_element_type=f32)
    asT[...] = lax.dot_general(ams_ref[...], h, (((0,), (1,)), ((), ())),
                               preferred_element_type=f32)
    adT[...] = lax.dot_general(amd_ref[...], h, (((0,), (1,)), ((), ())),
                               preferred_element_type=f32)
    bm = jnp.broadcast_to(jnp.max(a_s, axis=0)[:, None], (HEADS, 128))
    @pl.when(pl.program_id(0) == 0)
    def _():
        gmax[...] = bm
    @pl.when(pl.program_id(0) != 0)
    def _():
        gmax[...] = jnp.maximum(gmax[...], bm)


def _tk1(x, w, ams, amd, din, dcat, nchunk):
    hchunk = dcat // nchunk
    body = functools.partial(_tk1_body, nchunk)
    return pl.pallas_call(
        body,
        grid=(NB,),
        in_specs=[
            pl.BlockSpec((B, din), lambda i: (i, 0)),
            pl.BlockSpec((din, dcat), lambda i: (0, 0)),
            pl.BlockSpec((dcat, HEADS), lambda i: (0, 0)),
            pl.BlockSpec((dcat, HEADS), lambda i: (0, 0)),
        ],
        out_specs=[pl.BlockSpec((B, hchunk), lambda i: (i, 0))] * nchunk + [
            pl.BlockSpec((HEADS, B), lambda i: (0, i)),
            pl.BlockSpec((HEADS, B), lambda i: (0, i)),
            pl.BlockSpec((HEADS, 128), lambda i: (0, 0)),
        ],
        out_shape=[jax.ShapeDtypeStruct((NP, hchunk), f32)] * nchunk + [
            jax.ShapeDtypeStruct((HEADS, NP), f32),
            jax.ShapeDtypeStruct((HEADS, NP), f32),
            jax.ShapeDtypeStruct((HEADS, 128), f32),
        ],
    )(x, w, ams, amd)


def _tk2_body(asT_ref, adT_ref, gmax_ref, cT, wsT):
    g = gmax_ref[...][:, 0:1]
    a_s = asT_ref[...]
    a_d = adT_ref[...]
    c = _leaky(g + a_d)
    cT[...] = c
    wsT[...] = jnp.exp(_leaky(a_s + a_d) - c)


def _tk2(asT, adT, gmax):
    return pl.pallas_call(
        _tk2_body,
        out_shape=[jax.ShapeDtypeStruct((HEADS, NP), f32),
                   jax.ShapeDtypeStruct((HEADS, NP), f32)],
    )(asT, adT, gmax)


def _sc_body(C, hpc, epk, asT, adT, cT, *rest):
    nq = HEADS // hpc
    ppc = nq // 2
    hrefs = rest[:nq]
    orefs = rest[nq:2 * nq]
    drefs = rest[2 * nq:3 * nq]
    (tabs, tabd, tabc, rows2, wpad2, sd2, vbuf, vbuf16,
     out_sp, den_sp, gsem0, gsem1, ssem0, ssem1, isem0, isem1) = rest[3 * nq:]
    gsem = (gsem0, gsem1)
    ssem = (ssem0, ssem1)
    isem = (isem0, isem1)
    core = lax.axis_index("c")
    sid = lax.axis_index("s")
    G = C // 16
    gph = G // hpc
    row0 = sid * RPT

    @pl.loop(0, KE)
    def _(e):
        for b in range(2):
            wpad2[b, e, pl.ds(0, 16)] = jnp.zeros((16,), f32)

    for p in range(ppc):
        for cc in range(2):
            q = ppc * cc + p

            @pl.when(core == cc)
            def _(q=q):
                h_hbm = hrefs[q]
                o_hbm = orefs[q]
                d_hbm = drefs[q]
                j0 = hpc * q
                pltpu.sync_copy(asT.at[pl.ds(j0, hpc)], tabs)
                pltpu.sync_copy(adT.at[pl.ds(j0, hpc)], tabd)
                pltpu.sync_copy(cT.at[pl.ds(j0, hpc)], tabc)

                @pl.loop(0, RCH)
                def _(r):
                    for g in range(G):
                        vbuf[r, pl.ds(g * 16, 16)] = jnp.zeros((16,), f32)
                    vbuf16[r, pl.ds(0, 16)] = jnp.zeros((16,), f32)
                for t in range(RPT // RCH):
                    pltpu.sync_copy(vbuf, out_sp.at[pl.ds(row0 + t * RCH, RCH)])
                    pltpu.sync_copy(vbuf16, den_sp.at[pl.ds(row0 + t * RCH, RCH)])
                plsc.subcore_barrier()

                def drain_gather(b):
                    pltpu.make_async_copy(h_hbm.at[pl.ds(0, KE)],
                                          rows2.at[b], gsem[b]).wait()

                def drain_scatter(b):
                    pltpu.make_async_copy(h_hbm.at[pl.ds(0, KE)],
                                          rows2.at[b], ssem[b]).wait()
                    pltpu.make_async_copy(d_hbm.at[pl.ds(0, KE)],
                                          wpad2.at[b], ssem[b]).wait()

                cid0 = sid * NCH
                pltpu.sync_copy(epk.at[cid0], sd2.at[0])
                pltpu.async_copy(h_hbm.at[sd2.at[0, 0]], rows2.at[0], gsem[0])

                @pl.loop(0, NCH // 2)
                def _(k2):
                    for b in range(2):
                        nb = 1 - b
                        k = k2 * 2 + b
                        @pl.when(k + 1 < NCH)
                        def _():
                            @pl.when(k >= 1)
                            def _():
                                drain_scatter(nb)
                            pltpu.async_copy(epk.at[cid0 + k + 1], sd2.at[nb],
                                             isem[nb])
                        for v in range(KE // 16):
                            s16 = sd2[b, 0, pl.ds(v * 16, 16)]
                            d16 = sd2[b, 1, pl.ds(v * 16, 16)]
                            rowi = jnp.arange(16, dtype=i32) + (v * 16)
                            for dlt in range(hpc):
                                dv = jnp.full((16,), dlt, i32)
                                a_s = plsc.load_gather(tabs, [dv, s16])
                                a_d = plsc.load_gather(tabd, [dv, d16])
                                cb = plsc.load_gather(tabc, [dv, d16])
                                w = jnp.exp(_leaky(a_s + a_d) - cb)
                                plsc.store_scatter(wpad2.at[b], [rowi, dv], w)
                        drain_gather(b)
                        @pl.loop(0, KE, unroll=8)
                        def _(e):
                            wv = wpad2[b, e, pl.ds(0, 16)]
                            for g in range(G):
                                wj = wv[g // gph]
                                sl = pl.ds(g * 16, 16)
                                rows2[b, e, sl] = rows2[b, e, sl] * wj
                        pltpu.async_copy(rows2.at[b], out_sp.at[sd2.at[b, 1]],
                                         ssem[b], add=True)
                        pltpu.async_copy(wpad2.at[b], den_sp.at[sd2.at[b, 1]],
                                         ssem[b], add=True)
                        @pl.when(k + 1 < NCH)
                        def _():
                            pltpu.make_async_copy(epk.at[cid0], sd2.at[nb],
                                                  isem[nb]).wait()
                            pltpu.async_copy(h_hbm.at[sd2.at[nb, 0]],
                                             rows2.at[nb], gsem[nb])

                drain_scatter(0)
                drain_scatter(1)

                plsc.subcore_barrier()
                for t in range(RPT // RCH):
                    sl = pl.ds(row0 + t * RCH, RCH)
                    pltpu.sync_copy(out_sp.at[sl], vbuf)
                    pltpu.sync_copy(vbuf, o_hbm.at[sl])
                    pltpu.sync_copy(den_sp.at[sl], vbuf16)
                    pltpu.sync_copy(vbuf16, d_hbm.at[sl])


def _sc_layer(C, hpc, epk, asT, adT, cT, hs):
    nq = HEADS // hpc
    mesh = plsc.VectorSubcoreMesh(core_axis_name="c", subcore_axis_name="s")
    kern = pl.kernel(
        functools.partial(_sc_body, C, hpc),
        out_type=[jax.ShapeDtypeStruct((NP, C), f32)] * nq +
                 [jax.ShapeDtypeStruct((NP, 16), f32)] * nq,
        mesh=mesh,
        compiler_params=_SC_PARAMS,
        scratch_types=[
            pltpu.VMEM((hpc, NP), f32),
            pltpu.VMEM((hpc, NP), f32),
            pltpu.VMEM((hpc, NP), f32),
            pltpu.VMEM((2, KE, C), f32),
            pltpu.VMEM((2, KE, 16), f32),
            pltpu.VMEM((2, 2, KE), i32),
            pltpu.VMEM((RCH, C), f32),
            pltpu.VMEM((RCH, 16), f32),
            pltpu.VMEM_SHARED((NP, C), f32),
            pltpu.VMEM_SHARED((NP, 16), f32),
            pltpu.SemaphoreType.DMA,
            pltpu.SemaphoreType.DMA,
            pltpu.SemaphoreType.DMA,
            pltpu.SemaphoreType.DMA,
            pltpu.SemaphoreType.DMA,
            pltpu.SemaphoreType.DMA,
        ],
    )
    res = kern(epk, asT, adT, cT, *hs)
    return res[:nq], res[nq:]


def _tk3_body(*refs):
    (a0, a1, a2, a3, e0, e1, e2, e3, ws_ref,
     g0, g1, g2, g3, w2_ref, ams_ref, amd_ref, b1_ref) = refs[:17]
    orefs = refs[17:25]
    asT, adT, gmax = refs[25:]
    ws = ws_ref[...]
    accs = (a0, a1, a2, a3)
    dens = (e0, e1, e2, e3)
    hs = (g0, g1, g2, g3)
    pieces = []
    for q in range(4):
        acc = accs[q][...]
        den = dens[q][...]
        hq = hs[q][...]
        for dlt in range(2):
            j = 2 * q + dlt
            wsj = ws[j][:, None]
            num = acc[:, dlt * HID:(dlt + 1) * HID] + wsj * hq[:, dlt * HID:(dlt + 1) * HID]
            dj = den[:, dlt][:, None] + wsj + 1e-16
            pieces.append(num / dj)
    x1 = jnp.concatenate(pieces, axis=1) + b1_ref[...]
    h2 = jnp.dot(x1, w2_ref[...], preferred_element_type=f32)
    for q, oref in enumerate(orefs):
        oref[...] = h2[:, q * OUT:(q + 1) * OUT]
    a_s = jnp.dot(h2, ams_ref[...], preferred_element_type=f32)
    asT[...] = lax.dot_general(ams_ref[...], h2, (((0,), (1,)), ((), ())),
                               preferred_element_type=f32)
    adT[...] = lax.dot_general(amd_ref[...], h2, (((0,), (1,)), ((), ())),
                               preferred_element_type=f32)
    bm = jnp.broadcast_to(jnp.max(a_s, axis=0)[:, None], (HEADS, 128))
    @pl.when(pl.program_id(0) == 0)
    def _():
        gmax[...] = bm
    @pl.when(pl.program_id(0) != 0)
    def _():
        gmax[...] = jnp.maximum(gmax[...], bm)


def _tk3(acc1, den1, wsT1, h4, w2, ams2, amd2, b1):
    nblk = lambda s: pl.BlockSpec(s, lambda i: (i, 0))
    tblk = pl.BlockSpec((HEADS, B), lambda i: (0, i))
    return pl.pallas_call(
        _tk3_body,
        grid=(NB,),
        in_specs=[nblk((B, 32))] * 4 + [nblk((B, 16))] * 4 + [tblk] +
                 [nblk((B, 32))] * 4 + [
            pl.BlockSpec((D, 512), lambda i: (0, 0)),
            pl.BlockSpec((512, HEADS), lambda i: (0, 0)),
            pl.BlockSpec((512, HEADS), lambda i: (0, 0)),
            pl.BlockSpec((1, D), lambda i: (0, 0)),
        ],
        out_specs=[nblk((B, OUT))] * 8 + [tblk, tblk,
                   pl.BlockSpec((HEADS, 128), lambda i: (0, 0))],
        out_shape=[jax.ShapeDtypeStruct((NP, OUT), f32)] * 8 + [
            jax.ShapeDtypeStruct((HEADS, NP), f32),
            jax.ShapeDtypeStruct((HEADS, NP), f32),
            jax.ShapeDtypeStruct((HEADS, 128), f32),
        ],
    )(*acc1, *den1, wsT1, *h4, w2, ams2, amd2, b1)


def _tk5_body(*refs):
    accs = refs[:8]
    dens = refs[8:16]
    ws_ref = refs[16]
    hs = refs[17:25]
    bo_ref = refs[25]
    out = refs[26]
    ws = ws_ref[...]
    tot = jnp.zeros((B, OUT), f32)
    for j in range(HEADS):
        acc = accs[j][...]
        den = dens[j][...]
        hj = hs[j][...]
        wsj = ws[j][:, None]
        num = acc + wsj * hj
        dj = den[:, 0][:, None] + wsj + 1e-16
        tot = tot + num / dj
    y = tot * (1.0 / HEADS) + bo_ref[...]
    y = jnp.where(y > 0, y, jnp.exp(jnp.minimum(y, 0.0)) - 1.0)
    m = jnp.max(y, axis=1, keepdims=True)
    z = y - m
    out[...] = z - jnp.log(jnp.sum(jnp.exp(z), axis=1, keepdims=True))


def _tk5(acc2, den2, wsT2, h2c, bo):
    nblk = lambda s: pl.BlockSpec(s, lambda i: (i, 0))
    tblk = pl.BlockSpec((HEADS, B), lambda i: (0, i))
    return pl.pallas_call(
        _tk5_body,
        grid=(NB,),
        in_specs=[nblk((B, OUT))] * 8 + [nblk((B, 16))] * 8 + [tblk] +
                 [nblk((B, OUT))] * 8 + [pl.BlockSpec((1, OUT), lambda i: (0, 0))],
        out_specs=nblk((B, OUT)),
        out_shape=jax.ShapeDtypeStruct((NP, OUT), f32),
    )(*acc2, *den2, wsT2, *h2c, bo)


def kernel(x, edge_index, Ws, a_srcs, a_dsts, biases, W_out,
           a_src_out, a_dst_out, bias_out):
    x = x.astype(f32)
    xp = jnp.pad(x, ((0, NP - N), (0, 0)))
    epk = edge_index.astype(i32).reshape(2, E // KE, KE).transpose(1, 0, 2)

    eye8 = jnp.eye(HEADS, dtype=f32)
    W1 = Ws.astype(f32).transpose(1, 0, 2).reshape(D, HEADS * HID)
    ams1 = (eye8[:, None, :] * a_srcs.astype(f32)[:, :, None]).reshape(HEADS * HID, HEADS)
    amd1 = (eye8[:, None, :] * a_dsts.astype(f32)[:, :, None]).reshape(HEADS * HID, HEADS)
    ams2 = (eye8[:, None, :] * a_src_out.astype(f32)[:, :, None]).reshape(HEADS * OUT, HEADS)
    amd2 = (eye8[:, None, :] * a_dst_out.astype(f32)[:, :, None]).reshape(HEADS * OUT, HEADS)
    b1 = biases.astype(f32).reshape(1, HEADS * HID)
    bo = bias_out.astype(f32).reshape(1, OUT)

    *h4_1, asT1, adT1, gmax1 = _tk1(xp, W1, ams1, amd1, D, HEADS * HID, 4)
    cT1, wsT1 = _tk2(asT1, adT1, gmax1)
    acc1, den1 = _sc_layer(32, 2, epk, asT1, adT1, cT1, h4_1)

    *h2c, asT2, adT2, gmax2 = _tk3(acc1, den1, wsT1, h4_1, W_out.astype(f32),
                                   ams2, amd2, b1)
    cT2, wsT2 = _tk2(asT2, adT2, gmax2)
    acc2, den2 = _sc_layer(OUT, 1, epk, asT2, adT2, cT2, h2c)

    out = _tk5(acc2, den2, wsT2, h2c, bo)
    return out[:N]
